# Initial kernel scaffold; baseline (speedup 1.0000x reference)
import jax
import jax.numpy as jnp
from jax.experimental import pallas as pl


def kernel(x, edge_index, batch, W1, b1, W2, b2, Wc, bc):
    raise NotImplementedError("write your pallas kernel here")


# speedup vs baseline: 13.1772x; 13.1772x over previous
import functools

import jax
import jax.numpy as jnp
from jax import lax
from jax.experimental import pallas as pl
from jax.experimental.pallas import tpu as pltpu
from jax.experimental.pallas import tpu_sc as plsc

NUM_GRAPHS = 64
D = 128
CHUNK = 128
NC, NS = 2, 16
NW = NC * NS


def _sc_mesh():
    return plsc.VectorSubcoreMesh(core_axis_name="c", subcore_axis_name="s")


def _deg_body(R, CPT, dst_hbm, dega, degb, dst_v, ones_v, zbuf, shared, sem):
    c = lax.axis_index("c")
    s = lax.axis_index("s")
    wid = s * NC + c
    rows_per_tile = R // NS
    row0 = s * rows_per_tile

    zero16 = jnp.zeros((16,), jnp.float32)
    one16 = jnp.ones((16,), jnp.float32)
    for r in range(64):
        zbuf[r] = zero16
    for r in range(CHUNK):
        ones_v[r] = one16
    for k in range(rows_per_tile // 64):
        pltpu.sync_copy(zbuf, shared.at[pl.ds(row0 + k * 64, 64)])
    plsc.subcore_barrier()

    pltpu.sync_copy(dst_hbm.at[wid], dst_v)

    def step(j, carry):
        pltpu.sync_copy(ones_v, shared.at[dst_v.at[j]], add=True)
        return carry

    lax.fori_loop(0, CPT, step, 0)
    plsc.subcore_barrier()

    @pl.when(c == 0)
    def _():
        pltpu.sync_copy(shared.at[pl.ds(row0, rows_per_tile)],
                        dega.at[pl.ds(row0, rows_per_tile)])

    @pl.when(c == 1)
    def _():
        pltpu.sync_copy(shared.at[pl.ds(row0, rows_per_tile)],
                        degb.at[pl.ds(row0, rows_per_tile)])


def _agg_body(R, CPT, hs_hbm, src_hbm, dst_hbm, outa, outb,
              src_v, dst_v, rows_v, zbuf, shared, sem):
    c = lax.axis_index("c")
    s = lax.axis_index("s")
    wid = s * NC + c
    rows_per_tile = R // NS
    row0 = s * rows_per_tile

    zero16 = jnp.zeros((16,), jnp.float32)
    for r in range(16):
        for k in range(8):
            zbuf[r, pl.ds(k * 16, 16)] = zero16
    for k in range(rows_per_tile // 16):
        pltpu.sync_copy(zbuf, shared.at[pl.ds(row0 + k * 16, 16)])
    plsc.subcore_barrier()

    pltpu.sync_copy(src_hbm.at[wid], src_v)
    pltpu.sync_copy(dst_hbm.at[wid], dst_v)

    def step(j, carry):
        pltpu.async_copy(hs_hbm.at[src_v.at[j]], rows_v, sem).wait()
        pltpu.sync_copy(rows_v, shared.at[dst_v.at[j]], add=True)
        return carry

    lax.fori_loop(0, CPT, step, 0)
    plsc.subcore_barrier()

    @pl.when(c == 0)
    def _():
        pltpu.sync_copy(shared.at[pl.ds(row0, rows_per_tile)],
                        outa.at[pl.ds(row0, rows_per_tile)])

    @pl.when(c == 1)
    def _():
        pltpu.sync_copy(shared.at[pl.ds(row0, rows_per_tile)],
                        outb.at[pl.ds(row0, rows_per_tile)])


@functools.lru_cache(maxsize=None)
def _make_deg_kernel(R, CPT):
    return pl.kernel(
        functools.partial(_deg_body, R, CPT),
        out_type=(jax.ShapeDtypeStruct((R, 16), jnp.float32),
                  jax.ShapeDtypeStruct((R, 16), jnp.float32)),
        mesh=_sc_mesh(),
        scratch_types=[
            pltpu.VMEM((CPT, CHUNK), jnp.int32),
            pltpu.VMEM((CHUNK, 16), jnp.float32),
            pltpu.VMEM((64, 16), jnp.float32),
            pltpu.VMEM_SHARED((R, 16), jnp.float32),
            pltpu.SemaphoreType.DMA,
        ],
    )


@functools.lru_cache(maxsize=None)
def _make_agg_kernel(R, CPT):
    return pl.kernel(
        functools.partial(_agg_body, R, CPT),
        out_type=(jax.ShapeDtypeStruct((R, D), jnp.float32),
                  jax.ShapeDtypeStruct((R, D), jnp.float32)),
        mesh=_sc_mesh(),
        scratch_types=[
            pltpu.VMEM((CPT, CHUNK), jnp.int32),
            pltpu.VMEM((CPT, CHUNK), jnp.int32),
            pltpu.VMEM((CHUNK, D), jnp.float32),
            pltpu.VMEM((16, D), jnp.float32),
            pltpu.VMEM_SHARED((R, D), jnp.float32),
            pltpu.SemaphoreType.DMA,
        ],
    )


def _dinv_of(dega, degb):
    deg = dega[:, :1] + degb[:, :1] + 1.0
    return lax.rsqrt(deg)


def _prep_body(x_ref, w1_ref, dega_ref, degb_ref, hs_ref):
    dinv = _dinv_of(dega_ref[...], degb_ref[...])
    g = jnp.dot(x_ref[...], w1_ref[...], preferred_element_type=jnp.float32)
    hs_ref[...] = g * dinv


def _mid_body(u1a_ref, u1b_ref, hs1_ref, dega_ref, degb_ref, b1_ref, w2_ref,
              hs2_ref):
    dinv = _dinv_of(dega_ref[...], degb_ref[...])
    t = dinv * (u1a_ref[...] + u1b_ref[...] + hs1_ref[...]) + b1_ref[...]
    h = jnp.maximum(t, 0.0)
    g2 = jnp.dot(h, w2_ref[...], preferred_element_type=jnp.float32)
    hs2_ref[...] = g2 * dinv


def _fin_body(u2a_ref, u2b_ref, hs2_ref, dega_ref, degb_ref, b2_ref,
              bcol_ref, wc_ref, bc_ref, out_ref):
    dinv = _dinv_of(dega_ref[...], degb_ref[...])
    t = dinv * (u2a_ref[...] + u2b_ref[...] + hs2_ref[...]) + b2_ref[...]
    h2 = jnp.maximum(t, 0.0)
    gid = lax.broadcasted_iota(jnp.int32, bcol_ref.shape, 1)
    oh = jnp.where(bcol_ref[...] == gid, 1.0, 0.0)
    dn = (((0,), (0,)), ((), ()))
    pooled_s = lax.dot_general(oh, h2, dn, preferred_element_type=jnp.float32)
    cnt = lax.dot_general(oh, jnp.ones_like(h2), dn,
                          preferred_element_type=jnp.float32)
    pooled = pooled_s / jnp.maximum(cnt, 1.0)
    out_ref[...] = (jnp.dot(pooled, wc_ref[...],
                            preferred_element_type=jnp.float32) + bc_ref[...])


def _tc_call(body, n_out_rows, n_out_cols, args):
    return pl.pallas_call(
        body,
        out_shape=jax.ShapeDtypeStruct((n_out_rows, n_out_cols), jnp.float32),
    )(*args)


def kernel(x, edge_index, batch, W1, b1, W2, b2, Wc, bc):
    n = x.shape[0]
    e = edge_index.shape[1]
    R = ((n + 1 + 16 * NS - 1) // (16 * NS)) * (16 * NS)
    cpt = -(-e // (NW * CHUNK))
    ep = NW * cpt * CHUNK

    src = edge_index[0].astype(jnp.int32)
    dst = edge_index[1].astype(jnp.int32)
    src_p = jnp.concatenate(
        [src, jnp.zeros((ep - e,), jnp.int32)]).reshape(NW, cpt, CHUNK)
    dst_p = jnp.concatenate(
        [dst, jnp.full((ep - e,), n, jnp.int32)]).reshape(NW, cpt, CHUNK)

    x_p = jnp.zeros((R, D), jnp.float32).at[:n].set(x)
    bcol = jnp.full((R,), 100, jnp.int32).at[:n].set(batch.astype(jnp.int32))
    bcol = jnp.broadcast_to(bcol[:, None], (R, D))
    wc_p = jnp.zeros((D, D), jnp.float32).at[:, :Wc.shape[1]].set(Wc)
    bc_p = jnp.zeros((1, D), jnp.float32).at[0, :bc.shape[0]].set(bc)
    b1_p = b1.reshape(1, D)
    b2_p = b2.reshape(1, D)

    deg_k = _make_deg_kernel(R, cpt)
    agg_k = _make_agg_kernel(R, cpt)

    dega, degb = deg_k(dst_p)
    hs1 = _tc_call(_prep_body, R, D, (x_p, W1, dega, degb))
    u1a, u1b = agg_k(hs1, src_p, dst_p)
    hs2 = _tc_call(_mid_body, R, D, (u1a, u1b, hs1, dega, degb, b1_p, W2))
    u2a, u2b = agg_k(hs2, src_p, dst_p)
    out = _tc_call(_fin_body, NUM_GRAPHS + 64, D,
                   (u2a, u2b, hs2, dega, degb, b2_p, bcol, wc_p, bc_p))
    return out[:NUM_GRAPHS, :Wc.shape[1]]


# speedup vs baseline: 14.9635x; 1.1356x over previous
import functools

import jax
import jax.numpy as jnp
from jax import lax
from jax.experimental import pallas as pl
from jax.experimental.pallas import tpu as pltpu
from jax.experimental.pallas import tpu_sc as plsc

NUM_GRAPHS = 64
D = 128
CHUNK = 128
NC, NS = 2, 16
NW = NC * NS


def _sc_mesh():
    return plsc.VectorSubcoreMesh(core_axis_name="c", subcore_axis_name="s")


def _deg_body(R, CPT, dst_hbm, dega, degb, dst_v, ones_v, zbuf, shared, sem):
    c = lax.axis_index("c")
    s = lax.axis_index("s")
    wid = s * NC + c
    rows_per_tile = R // NS
    row0 = s * rows_per_tile

    zero16 = jnp.zeros((16,), jnp.float32)
    one16 = jnp.ones((16,), jnp.float32)
    for r in range(64):
        zbuf[r] = zero16
    for r in range(CHUNK):
        ones_v[r] = one16
    for k in range(rows_per_tile // 64):
        pltpu.sync_copy(zbuf, shared.at[pl.ds(row0 + k * 64, 64)])
    plsc.subcore_barrier()

    pltpu.sync_copy(dst_hbm.at[wid], dst_v)

    def step(j, carry):
        pltpu.sync_copy(ones_v, shared.at[dst_v.at[j]], add=True)
        return carry

    lax.fori_loop(0, CPT, step, 0)
    plsc.subcore_barrier()

    @pl.when(c == 0)
    def _():
        pltpu.sync_copy(shared.at[pl.ds(row0, rows_per_tile)],
                        dega.at[pl.ds(row0, rows_per_tile)])

    @pl.when(c == 1)
    def _():
        pltpu.sync_copy(shared.at[pl.ds(row0, rows_per_tile)],
                        degb.at[pl.ds(row0, rows_per_tile)])


def _unpack_chunk(packed_v, srcb, dstb, j, slot):
    mask = jnp.int32(0xFFFF)
    for k in range(CHUNK // 16):
        w = packed_v[j, pl.ds(k * 16, 16)]
        srcb[slot, pl.ds(k * 16, 16)] = jnp.bitwise_and(w, mask)
        dstb[slot, pl.ds(k * 16, 16)] = lax.shift_right_logical(
            w, jnp.int32(16))


def _agg_body(R, CPT, hs_hbm, pk_hbm, outa, outb,
              packed_v, srcb, dstb, rows_v, zbuf, shared, gsem, ssem):
    c = lax.axis_index("c")
    s = lax.axis_index("s")
    wid = s * NC + c
    rows_per_tile = R // NS
    row0 = s * rows_per_tile

    zero16 = jnp.zeros((16,), jnp.float32)
    for r in range(16):
        for k in range(8):
            zbuf[r, pl.ds(k * 16, 16)] = zero16
    for k in range(rows_per_tile // 16):
        pltpu.sync_copy(zbuf, shared.at[pl.ds(row0 + k * 16, 16)])
    plsc.subcore_barrier()

    pltpu.sync_copy(pk_hbm.at[wid], packed_v)

    _unpack_chunk(packed_v, srcb, dstb, 0, 0)
    pltpu.async_copy(hs_hbm.at[srcb.at[0]], rows_v.at[0], gsem)

    def step(j, carry):
        b2 = lax.rem(j, 2)
        nb2 = lax.rem(j + 1, 2)
        b3 = lax.rem(j, 3)
        nb3 = lax.rem(j + 1, 3)

        @pl.when(j > 0)
        def _():
            pltpu.make_async_copy(rows_v.at[nb2],
                                  shared.at[dstb.at[lax.rem(j - 1, 3)]],
                                  ssem).wait()

        @pl.when(j + 1 < CPT)
        def _():
            _unpack_chunk(packed_v, srcb, dstb, j + 1, nb3)
            pltpu.async_copy(hs_hbm.at[srcb.at[nb3]], rows_v.at[nb2], gsem)

        pltpu.make_async_copy(hs_hbm.at[srcb.at[b3]], rows_v.at[b2],
                              gsem).wait()
        pltpu.async_copy(rows_v.at[b2], shared.at[dstb.at[b3]], ssem,
                         add=True)
        return carry

    lax.fori_loop(0, CPT, step, 0)
    pltpu.make_async_copy(rows_v.at[lax.rem(CPT - 1, 2)],
                          shared.at[dstb.at[lax.rem(CPT - 1, 3)]],
                          ssem).wait()
    plsc.subcore_barrier()

    @pl.when(c == 0)
    def _():
        pltpu.sync_copy(shared.at[pl.ds(row0, rows_per_tile)],
                        outa.at[pl.ds(row0, rows_per_tile)])

    @pl.when(c == 1)
    def _():
        pltpu.sync_copy(shared.at[pl.ds(row0, rows_per_tile)],
                        outb.at[pl.ds(row0, rows_per_tile)])


@functools.lru_cache(maxsize=None)
def _make_deg_kernel(R, CPT):
    return pl.kernel(
        functools.partial(_deg_body, R, CPT),
        out_type=(jax.ShapeDtypeStruct((R, 16), jnp.float32),
                  jax.ShapeDtypeStruct((R, 16), jnp.float32)),
        mesh=_sc_mesh(),
        scratch_types=[
            pltpu.VMEM((CPT, CHUNK), jnp.int32),
            pltpu.VMEM((CHUNK, 16), jnp.float32),
            pltpu.VMEM((64, 16), jnp.float32),
            pltpu.VMEM_SHARED((R, 16), jnp.float32),
            pltpu.SemaphoreType.DMA,
        ],
    )


@functools.lru_cache(maxsize=None)
def _make_agg_kernel(R, CPT):
    return pl.kernel(
        functools.partial(_agg_body, R, CPT),
        out_type=(jax.ShapeDtypeStruct((R, D), jnp.float32),
                  jax.ShapeDtypeStruct((R, D), jnp.float32)),
        mesh=_sc_mesh(),
        scratch_types=[
            pltpu.VMEM((CPT, CHUNK), jnp.int32),
            pltpu.VMEM((3, CHUNK), jnp.int32),
            pltpu.VMEM((3, CHUNK), jnp.int32),
            pltpu.VMEM((2, CHUNK, D), jnp.float32),
            pltpu.VMEM((16, D), jnp.float32),
            pltpu.VMEM_SHARED((R, D), jnp.float32),
            pltpu.SemaphoreType.DMA,
            pltpu.SemaphoreType.DMA,
        ],
    )


def _dinv_of(dega, degb):
    deg = dega[:, :1] + degb[:, :1] + 1.0
    return lax.rsqrt(deg)


def _prep_body(x_ref, w1_ref, dega_ref, degb_ref, hs_ref):
    dinv = _dinv_of(dega_ref[...], degb_ref[...])
    g = jnp.dot(x_ref[...], w1_ref[...], preferred_element_type=jnp.float32)
    hs_ref[...] = g * dinv


def _mid_body(u1a_ref, u1b_ref, hs1_ref, dega_ref, degb_ref, b1_ref, w2_ref,
              hs2_ref):
    dinv = _dinv_of(dega_ref[...], degb_ref[...])
    t = dinv * (u1a_ref[...] + u1b_ref[...] + hs1_ref[...]) + b1_ref[...]
    h = jnp.maximum(t, 0.0)
    g2 = jnp.dot(h, w2_ref[...], preferred_element_type=jnp.float32)
    hs2_ref[...] = g2 * dinv


def _fin_body(u2a_ref, u2b_ref, hs2_ref, dega_ref, degb_ref, b2_ref,
              bcol_ref, wc_ref, bc_ref, out_ref):
    dinv = _dinv_of(dega_ref[...], degb_ref[...])
    t = dinv * (u2a_ref[...] + u2b_ref[...] + hs2_ref[...]) + b2_ref[...]
    h2 = jnp.maximum(t, 0.0)
    gid = lax.broadcasted_iota(jnp.int32, bcol_ref.shape, 1)
    oh = jnp.where(bcol_ref[...] == gid, 1.0, 0.0)
    dn = (((0,), (0,)), ((), ()))
    pooled_s = lax.dot_general(oh, h2, dn, preferred_element_type=jnp.float32)
    cnt = lax.dot_general(oh, jnp.ones_like(h2), dn,
                          preferred_element_type=jnp.float32)
    pooled = pooled_s / jnp.maximum(cnt, 1.0)
    out_ref[...] = (jnp.dot(pooled, wc_ref[...],
                            preferred_element_type=jnp.float32) + bc_ref[...])


def _tc_call(body, n_out_rows, n_out_cols, args):
    return pl.pallas_call(
        body,
        out_shape=jax.ShapeDtypeStruct((n_out_rows, n_out_cols), jnp.float32),
    )(*args)


def kernel(x, edge_index, batch, W1, b1, W2, b2, Wc, bc):
    n = x.shape[0]
    e = edge_index.shape[1]
    R = ((n + 1 + 16 * NS - 1) // (16 * NS)) * (16 * NS)
    cpt = -(-e // (NW * CHUNK))
    ep = NW * cpt * CHUNK

    src = edge_index[0].astype(jnp.int32)
    dst = edge_index[1].astype(jnp.int32)
    src_p = jnp.concatenate([src, jnp.zeros((ep - e,), jnp.int32)])
    dst_p = jnp.concatenate([dst, jnp.full((ep - e,), n, jnp.int32)])
    packed = (src_p | (dst_p << 16)).reshape(NW, cpt, CHUNK)
    dst_p = dst_p.reshape(NW, cpt, CHUNK)

    x_p = jnp.zeros((R, D), jnp.float32).at[:n].set(x)
    bcol = jnp.full((R,), 100, jnp.int32).at[:n].set(batch.astype(jnp.int32))
    bcol = jnp.broadcast_to(bcol[:, None], (R, D))
    wc_p = jnp.zeros((D, D), jnp.float32).at[:, :Wc.shape[1]].set(Wc)
    bc_p = jnp.zeros((1, D), jnp.float32).at[0, :bc.shape[0]].set(bc)
    b1_p = b1.reshape(1, D)
    b2_p = b2.reshape(1, D)

    deg_k = _make_deg_kernel(R, cpt)
    agg_k = _make_agg_kernel(R, cpt)

    dega, degb = deg_k(dst_p)
    hs1 = _tc_call(_prep_body, R, D, (x_p, W1, dega, degb))
    u1a, u1b = agg_k(hs1, packed)
    hs2 = _tc_call(_mid_body, R, D, (u1a, u1b, hs1, dega, degb, b1_p, W2))
    u2a, u2b = agg_k(hs2, packed)
    out = _tc_call(_fin_body, NUM_GRAPHS + 64, D,
                   (u2a, u2b, hs2, dega, degb, b2_p, bcol, wc_p, bc_p))
    return out[:NUM_GRAPHS, :Wc.shape[1]]


# speedup vs baseline: 32.6821x; 2.1841x over previous
import functools

import jax
import jax.numpy as jnp
from jax import lax
from jax.experimental import pallas as pl
from jax.experimental.pallas import tpu as pltpu
from jax.experimental.pallas import tpu_sc as plsc

NUM_GRAPHS = 64
D = 128
CHUNK = 128
NC, NS = 2, 16
NW = NC * NS


def _sc_mesh():
    return plsc.VectorSubcoreMesh(core_axis_name="c", subcore_axis_name="s")


def _deg_body(R, CPT, dst_hbm, dega, degb, dst_v, ones_v, zbuf, shared, sem):
    c = lax.axis_index("c")
    s = lax.axis_index("s")
    wid = s * NC + c
    rows_per_tile = R // NS
    row0 = s * rows_per_tile

    zero16 = jnp.zeros((16,), jnp.float32)
    one16 = jnp.ones((16,), jnp.float32)
    for r in range(64):
        zbuf[r] = zero16
    for r in range(CHUNK):
        ones_v[r] = one16
    for k in range(rows_per_tile // 64):
        pltpu.sync_copy(zbuf, shared.at[pl.ds(row0 + k * 64, 64)])
    plsc.subcore_barrier()

    pltpu.sync_copy(dst_hbm.at[wid], dst_v)

    def step(j, carry):
        pltpu.sync_copy(ones_v, shared.at[dst_v.at[j]], add=True)
        return carry

    lax.fori_loop(0, CPT, step, 0)
    plsc.subcore_barrier()

    @pl.when(c == 0)
    def _():
        pltpu.sync_copy(shared.at[pl.ds(row0, rows_per_tile)],
                        dega.at[pl.ds(row0, rows_per_tile)])

    @pl.when(c == 1)
    def _():
        pltpu.sync_copy(shared.at[pl.ds(row0, rows_per_tile)],
                        degb.at[pl.ds(row0, rows_per_tile)])


def _unpack_chunk(packed_v, srcb, dstb, j, slot):
    mask = jnp.int32(0xFFFF)
    for k in range(CHUNK // 16):
        w = packed_v[j, pl.ds(k * 16, 16)]
        srcb[slot, pl.ds(k * 16, 16)] = jnp.bitwise_and(w, mask)
        dstb[slot, pl.ds(k * 16, 16)] = lax.shift_right_logical(
            w, jnp.int32(16))


def _agg_body(R, CPT, hs_hbm, pk_hbm, outa, outb,
              packed_v, srcb, dstb, rows_v, zbuf, shared, gsem, ssem):
    c = lax.axis_index("c")
    s = lax.axis_index("s")
    wid = s * NC + c
    rows_per_tile = R // NS
    row0 = s * rows_per_tile

    zero16 = jnp.zeros((16,), jnp.float32)
    for r in range(16):
        for k in range(8):
            zbuf[r, pl.ds(k * 16, 16)] = zero16
    for k in range(rows_per_tile // 16):
        pltpu.sync_copy(zbuf, shared.at[pl.ds(row0 + k * 16, 16)])
    plsc.subcore_barrier()

    pltpu.sync_copy(pk_hbm.at[wid], packed_v)

    _unpack_chunk(packed_v, srcb, dstb, 0, 0)
    pltpu.async_copy(hs_hbm.at[srcb.at[0]], rows_v.at[0], gsem)

    def step(j, carry):
        b2 = lax.rem(j, 2)
        nb2 = lax.rem(j + 1, 2)
        b3 = lax.rem(j, 3)
        nb3 = lax.rem(j + 1, 3)

        @pl.when(j > 0)
        def _():
            pltpu.make_async_copy(rows_v.at[nb2],
                                  shared.at[dstb.at[lax.rem(j - 1, 3)]],
                                  ssem).wait()

        @pl.when(j + 1 < CPT)
        def _():
            _unpack_chunk(packed_v, srcb, dstb, j + 1, nb3)
            pltpu.async_copy(hs_hbm.at[srcb.at[nb3]], rows_v.at[nb2], gsem)

        pltpu.make_async_copy(hs_hbm.at[srcb.at[b3]], rows_v.at[b2],
                              gsem).wait()
        pltpu.async_copy(rows_v.at[b2], shared.at[dstb.at[b3]], ssem,
                         add=True)
        return carry

    lax.fori_loop(0, CPT, step, 0)
    pltpu.make_async_copy(rows_v.at[lax.rem(CPT - 1, 2)],
                          shared.at[dstb.at[lax.rem(CPT - 1, 3)]],
                          ssem).wait()
    plsc.subcore_barrier()

    @pl.when(c == 0)
    def _():
        pltpu.sync_copy(shared.at[pl.ds(row0, rows_per_tile)],
                        outa.at[pl.ds(row0, rows_per_tile)])

    @pl.when(c == 1)
    def _():
        pltpu.sync_copy(shared.at[pl.ds(row0, rows_per_tile)],
                        outb.at[pl.ds(row0, rows_per_tile)])


@functools.lru_cache(maxsize=None)
def _make_deg_kernel(R, CPT):
    return pl.kernel(
        functools.partial(_deg_body, R, CPT),
        out_type=(jax.ShapeDtypeStruct((R, 16), jnp.float32),
                  jax.ShapeDtypeStruct((R, 16), jnp.float32)),
        mesh=_sc_mesh(),
        scratch_types=[
            pltpu.VMEM((CPT, CHUNK), jnp.int32),
            pltpu.VMEM((CHUNK, 16), jnp.float32),
            pltpu.VMEM((64, 16), jnp.float32),
            pltpu.VMEM_SHARED((R, 16), jnp.float32),
            pltpu.SemaphoreType.DMA,
        ],
    )


@functools.lru_cache(maxsize=None)
def _make_agg_kernel(R, CPT):
    return pl.kernel(
        functools.partial(_agg_body, R, CPT),
        out_type=(jax.ShapeDtypeStruct((R, D), jnp.float32),
                  jax.ShapeDtypeStruct((R, D), jnp.float32)),
        mesh=_sc_mesh(),
        scratch_types=[
            pltpu.VMEM((CPT, CHUNK), jnp.int32),
            pltpu.VMEM((3, CHUNK), jnp.int32),
            pltpu.VMEM((3, CHUNK), jnp.int32),
            pltpu.VMEM((2, CHUNK, D), jnp.float32),
            pltpu.VMEM((16, D), jnp.float32),
            pltpu.VMEM_SHARED((R, D), jnp.float32),
            pltpu.SemaphoreType.DMA,
            pltpu.SemaphoreType.DMA,
        ],
    )


def _dinv_of(dega, degb):
    deg = dega[:, :1] + degb[:, :1] + 1.0
    return lax.rsqrt(deg)


def _prep_body(x_ref, w1_ref, dega_ref, degb_ref, hs_ref):
    dinv = _dinv_of(dega_ref[...], degb_ref[...])
    g = jnp.dot(x_ref[...], w1_ref[...], preferred_element_type=jnp.float32)
    hs_ref[...] = g * dinv


def _mid_body(u1a_ref, u1b_ref, hs1_ref, dega_ref, degb_ref, b1_ref, w2_ref,
              hs2_ref):
    dinv = _dinv_of(dega_ref[...], degb_ref[...])
    t = dinv * (u1a_ref[...] + u1b_ref[...] + hs1_ref[...]) + b1_ref[...]
    h = jnp.maximum(t, 0.0)
    g2 = jnp.dot(h, w2_ref[...], preferred_element_type=jnp.float32)
    hs2_ref[...] = g2 * dinv


def _fin_body(u2a_ref, u2b_ref, hs2_ref, dega_ref, degb_ref, b2_ref,
              bcol_ref, wc_ref, bc_ref, out_ref):
    dinv = _dinv_of(dega_ref[...], degb_ref[...])
    t = dinv * (u2a_ref[...] + u2b_ref[...] + hs2_ref[...]) + b2_ref[...]
    h2 = jnp.maximum(t, 0.0)
    gid = lax.broadcasted_iota(jnp.int32, bcol_ref.shape, 1)
    oh = jnp.where(bcol_ref[...] == gid, 1.0, 0.0)
    dn = (((0,), (0,)), ((), ()))
    pooled_s = lax.dot_general(oh, h2, dn, preferred_element_type=jnp.float32)
    cnt = lax.dot_general(oh, jnp.ones_like(h2), dn,
                          preferred_element_type=jnp.float32)
    pooled = pooled_s / jnp.maximum(cnt, 1.0)
    out_ref[...] = (jnp.dot(pooled, wc_ref[...],
                            preferred_element_type=jnp.float32) + bc_ref[...])


def _tc_call(body, n_out_rows, n_out_cols, args):
    return pl.pallas_call(
        body,
        out_shape=jax.ShapeDtypeStruct((n_out_rows, n_out_cols), jnp.float32),
    )(*args)


def kernel(x, edge_index, batch, W1, b1, W2, b2, Wc, bc):
    n = x.shape[0]
    e = edge_index.shape[1]
    R = ((n + 1 + 16 * NS - 1) // (16 * NS)) * (16 * NS)
    cpt = -(-e // (NW * CHUNK))
    ep = NW * cpt * CHUNK

    src = edge_index[0].astype(jnp.int32)
    dst = edge_index[1].astype(jnp.int32)
    pad_i = jnp.arange(ep - e, dtype=jnp.int32)
    src_p = jnp.concatenate([src, pad_i % jnp.int32(n)])
    dst_p = jnp.concatenate([dst, jnp.int32(n) + pad_i % jnp.int32(R - n)])
    packed = (src_p | (dst_p << 16)).reshape(NW, cpt, CHUNK)
    dst_p = dst_p.reshape(NW, cpt, CHUNK)

    x_p = jnp.zeros((R, D), jnp.float32).at[:n].set(x)
    bcol = jnp.full((R,), 100, jnp.int32).at[:n].set(batch.astype(jnp.int32))
    bcol = jnp.broadcast_to(bcol[:, None], (R, D))
    wc_p = jnp.zeros((D, D), jnp.float32).at[:, :Wc.shape[1]].set(Wc)
    bc_p = jnp.zeros((1, D), jnp.float32).at[0, :bc.shape[0]].set(bc)
    b1_p = b1.reshape(1, D)
    b2_p = b2.reshape(1, D)

    deg_k = _make_deg_kernel(R, cpt)
    agg_k = _make_agg_kernel(R, cpt)

    dega, degb = deg_k(dst_p)
    hs1 = _tc_call(_prep_body, R, D, (x_p, W1, dega, degb))
    u1a, u1b = agg_k(hs1, packed)
    hs2 = _tc_call(_mid_body, R, D, (u1a, u1b, hs1, dega, degb, b1_p, W2))
    u2a, u2b = agg_k(hs2, packed)
    out = _tc_call(_fin_body, NUM_GRAPHS + 64, D,
                   (u2a, u2b, hs2, dega, degb, b2_p, bcol, wc_p, bc_p))
    return out[:NUM_GRAPHS, :Wc.shape[1]]


# speedup vs baseline: 33.5450x; 1.0264x over previous
import functools

import jax
import jax.numpy as jnp
from jax import lax
from jax.experimental import pallas as pl
from jax.experimental.pallas import tpu as pltpu
from jax.experimental.pallas import tpu_sc as plsc

NUM_GRAPHS = 64
D = 128
CHUNK = 128
NC, NS = 2, 16
NW = NC * NS


def _sc_mesh():
    return plsc.VectorSubcoreMesh(core_axis_name="c", subcore_axis_name="s")


def _deg_body(R, CPT, dst_hbm, dega, degb, dst_v, ones_v, zbuf, shared, sem):
    c = lax.axis_index("c")
    s = lax.axis_index("s")
    wid = s * NC + c
    rows_per_tile = R // NS
    row0 = s * rows_per_tile

    idx_load = pltpu.async_copy(dst_hbm.at[wid], dst_v, sem)

    zero16 = jnp.zeros((16,), jnp.float32)
    one16 = jnp.ones((16,), jnp.float32)
    for r in range(64):
        zbuf[r] = zero16
    for r in range(CHUNK):
        ones_v[r] = one16
    for k in range(rows_per_tile // 64):
        pltpu.sync_copy(zbuf, shared.at[pl.ds(row0 + k * 64, 64)])
    plsc.subcore_barrier()
    idx_load.wait()

    def step(j, carry):
        pltpu.sync_copy(ones_v, shared.at[dst_v.at[j]], add=True)
        return carry

    lax.fori_loop(0, CPT, step, 0)
    plsc.subcore_barrier()

    @pl.when(c == 0)
    def _():
        pltpu.sync_copy(shared.at[pl.ds(row0, rows_per_tile)],
                        dega.at[pl.ds(row0, rows_per_tile)])

    @pl.when(c == 1)
    def _():
        pltpu.sync_copy(shared.at[pl.ds(row0, rows_per_tile)],
                        degb.at[pl.ds(row0, rows_per_tile)])


def _unpack_chunk(packed_v, srcb, dstb, j, slot):
    mask = jnp.int32(0xFFFF)
    for k in range(CHUNK // 16):
        w = packed_v[j, pl.ds(k * 16, 16)]
        srcb[slot, pl.ds(k * 16, 16)] = jnp.bitwise_and(w, mask)
        dstb[slot, pl.ds(k * 16, 16)] = lax.shift_right_logical(
            w, jnp.int32(16))


def _agg_body(R, CPT, hs_hbm, pk_hbm, outa, outb,
              packed_v, srcb, dstb, rows_v, zbuf, shared, gsem, ssem):
    c = lax.axis_index("c")
    s = lax.axis_index("s")
    wid = s * NC + c
    rows_per_tile = R // NS
    row0 = s * rows_per_tile

    idx_load = pltpu.async_copy(pk_hbm.at[wid], packed_v, gsem)

    zero16 = jnp.zeros((16,), jnp.float32)
    for r in range(16):
        for k in range(8):
            zbuf[r, pl.ds(k * 16, 16)] = zero16
    for k in range(rows_per_tile // 16):
        pltpu.sync_copy(zbuf, shared.at[pl.ds(row0 + k * 16, 16)])
    plsc.subcore_barrier()
    idx_load.wait()

    _unpack_chunk(packed_v, srcb, dstb, 0, 0)
    pltpu.async_copy(hs_hbm.at[srcb.at[0]], rows_v.at[0], gsem)

    def step(j, carry):
        b2 = lax.rem(j, 2)
        nb2 = lax.rem(j + 1, 2)
        b3 = lax.rem(j, 3)
        nb3 = lax.rem(j + 1, 3)

        @pl.when(j > 0)
        def _():
            pltpu.make_async_copy(rows_v.at[nb2],
                                  shared.at[dstb.at[lax.rem(j - 1, 3)]],
                                  ssem).wait()

        @pl.when(j + 1 < CPT)
        def _():
            _unpack_chunk(packed_v, srcb, dstb, j + 1, nb3)
            pltpu.async_copy(hs_hbm.at[srcb.at[nb3]], rows_v.at[nb2], gsem)

        pltpu.make_async_copy(hs_hbm.at[srcb.at[b3]], rows_v.at[b2],
                              gsem).wait()
        pltpu.async_copy(rows_v.at[b2], shared.at[dstb.at[b3]], ssem,
                         add=True)
        return carry

    lax.fori_loop(0, CPT, step, 0)
    pltpu.make_async_copy(rows_v.at[lax.rem(CPT - 1, 2)],
                          shared.at[dstb.at[lax.rem(CPT - 1, 3)]],
                          ssem).wait()
    plsc.subcore_barrier()

    @pl.when(c == 0)
    def _():
        pltpu.sync_copy(shared.at[pl.ds(row0, rows_per_tile)],
                        outa.at[pl.ds(row0, rows_per_tile)])

    @pl.when(c == 1)
    def _():
        pltpu.sync_copy(shared.at[pl.ds(row0, rows_per_tile)],
                        outb.at[pl.ds(row0, rows_per_tile)])


@functools.lru_cache(maxsize=None)
def _make_deg_kernel(R, CPT):
    return pl.kernel(
        functools.partial(_deg_body, R, CPT),
        out_type=(jax.ShapeDtypeStruct((R, 16), jnp.float32),
                  jax.ShapeDtypeStruct((R, 16), jnp.float32)),
        mesh=_sc_mesh(),
        scratch_types=[
            pltpu.VMEM((CPT, CHUNK), jnp.int32),
            pltpu.VMEM((CHUNK, 16), jnp.float32),
            pltpu.VMEM((64, 16), jnp.float32),
            pltpu.VMEM_SHARED((R, 16), jnp.float32),
            pltpu.SemaphoreType.DMA,
        ],
    )


@functools.lru_cache(maxsize=None)
def _make_agg_kernel(R, CPT):
    return pl.kernel(
        functools.partial(_agg_body, R, CPT),
        out_type=(jax.ShapeDtypeStruct((R, D), jnp.float32),
                  jax.ShapeDtypeStruct((R, D), jnp.float32)),
        mesh=_sc_mesh(),
        scratch_types=[
            pltpu.VMEM((CPT, CHUNK), jnp.int32),
            pltpu.VMEM((3, CHUNK), jnp.int32),
            pltpu.VMEM((3, CHUNK), jnp.int32),
            pltpu.VMEM((2, CHUNK, D), jnp.float32),
            pltpu.VMEM((16, D), jnp.float32),
            pltpu.VMEM_SHARED((R, D), jnp.float32),
            pltpu.SemaphoreType.DMA,
            pltpu.SemaphoreType.DMA,
        ],
    )


def _dinv_of(dega, degb):
    deg = dega[:, :1] + degb[:, :1] + 1.0
    return lax.rsqrt(deg)


def _prep_body(x_ref, w1_ref, dega_ref, degb_ref, hs_ref):
    dinv = _dinv_of(dega_ref[...], degb_ref[...])
    g = jnp.dot(x_ref[...], w1_ref[...], preferred_element_type=jnp.float32)
    hs_ref[...] = g * dinv


def _mid_body(u1a_ref, u1b_ref, hs1_ref, dega_ref, degb_ref, b1_ref, w2_ref,
              hs2_ref):
    dinv = _dinv_of(dega_ref[...], degb_ref[...])
    t = dinv * (u1a_ref[...] + u1b_ref[...] + hs1_ref[...]) + b1_ref[...]
    h = jnp.maximum(t, 0.0)
    g2 = jnp.dot(h, w2_ref[...], preferred_element_type=jnp.float32)
    hs2_ref[...] = g2 * dinv


def _fin_body(u2a_ref, u2b_ref, hs2_ref, dega_ref, degb_ref, b2_ref,
              bcol_ref, wc_ref, bc_ref, out_ref):
    dinv = _dinv_of(dega_ref[...], degb_ref[...])
    t = dinv * (u2a_ref[...] + u2b_ref[...] + hs2_ref[...]) + b2_ref[...]
    h2 = jnp.maximum(t, 0.0)
    gid = lax.broadcasted_iota(jnp.int32, bcol_ref.shape, 1)
    oh = jnp.where(bcol_ref[...] == gid, 1.0, 0.0)
    dn = (((0,), (0,)), ((), ()))
    pooled_s = lax.dot_general(oh, h2, dn, preferred_element_type=jnp.float32)
    cnt = lax.dot_general(oh, jnp.ones_like(h2), dn,
                          preferred_element_type=jnp.float32)
    pooled = pooled_s / jnp.maximum(cnt, 1.0)
    out_ref[...] = (jnp.dot(pooled, wc_ref[...],
                            preferred_element_type=jnp.float32) + bc_ref[...])


def _tc_call(body, n_out_rows, n_out_cols, args):
    return pl.pallas_call(
        body,
        out_shape=jax.ShapeDtypeStruct((n_out_rows, n_out_cols), jnp.float32),
    )(*args)


def kernel(x, edge_index, batch, W1, b1, W2, b2, Wc, bc):
    n = x.shape[0]
    e = edge_index.shape[1]
    R = ((n + 1 + 16 * NS - 1) // (16 * NS)) * (16 * NS)
    cpt = -(-e // (NW * CHUNK))
    ep = NW * cpt * CHUNK

    src = edge_index[0].astype(jnp.int32)
    dst = edge_index[1].astype(jnp.int32)
    pad_i = jnp.arange(ep - e, dtype=jnp.int32)
    src_p = jnp.concatenate([src, pad_i % jnp.int32(n)])
    dst_p = jnp.concatenate([dst, jnp.int32(n) + pad_i % jnp.int32(R - n)])
    packed = (src_p | (dst_p << 16)).reshape(NW, cpt, CHUNK)
    dst_p = dst_p.reshape(NW, cpt, CHUNK)

    x_p = jnp.zeros((R, D), jnp.float32).at[:n].set(x)
    bcol = jnp.full((R,), 100, jnp.int32).at[:n].set(batch.astype(jnp.int32))
    bcol = jnp.broadcast_to(bcol[:, None], (R, D))
    wc_p = jnp.zeros((D, D), jnp.float32).at[:, :Wc.shape[1]].set(Wc)
    bc_p = jnp.zeros((1, D), jnp.float32).at[0, :bc.shape[0]].set(bc)
    b1_p = b1.reshape(1, D)
    b2_p = b2.reshape(1, D)

    deg_k = _make_deg_kernel(R, cpt)
    agg_k = _make_agg_kernel(R, cpt)

    dega, degb = deg_k(dst_p)
    hs1 = _tc_call(_prep_body, R, D, (x_p, W1, dega, degb))
    u1a, u1b = agg_k(hs1, packed)
    hs2 = _tc_call(_mid_body, R, D, (u1a, u1b, hs1, dega, degb, b1_p, W2))
    u2a, u2b = agg_k(hs2, packed)
    out = _tc_call(_fin_body, NUM_GRAPHS + 64, D,
                   (u2a, u2b, hs2, dega, degb, b2_p, bcol, wc_p, bc_p))
    return out[:NUM_GRAPHS, :Wc.shape[1]]
